# baseline (speedup 1.0000x reference)
import functools

import jax
import jax.numpy as jnp
from jax import lax
from jax.experimental import pallas as pl
from jax.experimental.pallas import tpu as pltpu
from jax.experimental.pallas import tpu_sc as plsc

N = 10000
NP = 10240
F_IN = 128
D = 64
NCLS = 40
NCLSP = 128

NC = 2
NS = 16
NW = NC * NS
L = 16

C = 128
NBUF = 4
E_REAL = 320000 + N
CHUNKS = NBUF * (-(-E_REAL // (NW * C * NBUF)))
EPT = CHUNKS * C
EP = EPT * NW

SLICE_PT = NP // NS

_mesh = plsc.VectorSubcoreMesh(
    core_axis_name="c", subcore_axis_name="s", num_cores=NC, num_subcores=NS)
_sc_params = pltpu.CompilerParams(
    needs_layout_passes=False, use_tc_tiling_on_sc=False)


def _worker_id():
  return lax.axis_index("s") * NC + lax.axis_index("c")


@functools.partial(
    pl.kernel,
    out_type=jax.ShapeDtypeStruct((NW, NP), jnp.float32),
    mesh=_mesh,
    scratch_types=[
        pltpu.VMEM((CHUNKS, C), jnp.int32),
        pltpu.VMEM((CHUNKS, C), jnp.float32),
        pltpu.VMEM((NP,), jnp.float32),
    ],
    compiler_params=_sc_params,
)
def _deg_kernel(col_hbm, ew_hbm, deg_hbm, colb, ewb, degl):
  wid = _worker_id()

  def zero_body(i, carry):
    degl[pl.ds(i * L, L)] = jnp.zeros((L,), jnp.float32)
    return carry
  lax.fori_loop(0, NP // L, zero_body, 0)

  pltpu.sync_copy(col_hbm.at[wid], colb)
  pltpu.sync_copy(ew_hbm.at[wid], ewb)

  def chunk_body(i, carry):
    for g in range(C // L):
      cv = colb[i, pl.ds(g * L, L)]
      ev = ewb[i, pl.ds(g * L, L)]
      plsc.addupdate_scatter(degl, [cv], ev)
    return carry
  lax.fori_loop(0, CHUNKS, chunk_body, 0)

  pltpu.sync_copy(degl, deg_hbm.at[wid])


def _make_spmm(compute_norm):
  acc_type = jax.ShapeDtypeStruct((NC, NP, D), jnp.float32)
  if compute_norm:
    out_types = [acc_type,
                 jax.ShapeDtypeStruct((NW, CHUNKS, C), jnp.float32)]
  else:
    out_types = acc_type
  scratch = [
      pltpu.VMEM((CHUNKS, C), jnp.int32),
      pltpu.VMEM((CHUNKS, C), jnp.int32),
      pltpu.VMEM((CHUNKS, C), jnp.float32),
      [pltpu.VMEM((C, D), jnp.float32) for _ in range(NBUF)],
      pltpu.VMEM_SHARED((NP, D), jnp.float32),
      [pltpu.SemaphoreType.DMA for _ in range(NBUF)],
      [pltpu.SemaphoreType.DMA for _ in range(NBUF)],
  ]
  if compute_norm:
    scratch.insert(3, pltpu.VMEM((CHUNKS, C), jnp.float32))
    scratch.insert(0, pltpu.VMEM((NP,), jnp.float32))

  def body(*refs):
    if compute_norm:
      (row_hbm, col_hbm, ew_hbm, dis_hbm, x_hbm, out_hbm, norm_hbm,
       dis_l, rowb, colb, nb, ewb, rows, acc, gsem, ssem) = refs
    else:
      (row_hbm, col_hbm, norm_in_hbm, x_hbm, out_hbm,
       rowb, colb, nb, rows, acc, gsem, ssem) = refs
    cid = lax.axis_index("c")
    sid = lax.axis_index("s")
    wid = sid * NC + cid

    pltpu.sync_copy(row_hbm.at[wid], rowb)
    pltpu.sync_copy(col_hbm.at[wid], colb)
    if compute_norm:
      pltpu.sync_copy(ew_hbm.at[wid], ewb)
      pltpu.sync_copy(dis_hbm, dis_l)
    else:
      pltpu.sync_copy(norm_in_hbm.at[wid], nb)

    for r in range(C):
      for q in range(D // L):
        rows[0][r, pl.ds(q * L, L)] = jnp.zeros((L,), jnp.float32)
    for j in range(SLICE_PT // C):
      pltpu.sync_copy(rows[0], acc.at[pl.ds(sid * SLICE_PT + j * C, C)])
    plsc.subcore_barrier()

    def gather(it, b):
      return pltpu.async_copy(x_hbm.at[colb.at[it]], rows[b], gsem[b])

    def scatter(it, b):
      return pltpu.async_copy(rows[b], acc.at[rowb.at[it]], ssem[b],
                              add=True)

    for b in range(NBUF - 1):
      gather(b, b)

    def process(it, b):
      pltpu.make_async_copy(x_hbm.at[colb.at[it]], rows[b], gsem[b]).wait()
      if compute_norm:
        for g in range(C // L):
          rv = rowb[it, pl.ds(g * L, L)]
          cv = colb[it, pl.ds(g * L, L)]
          dr = plsc.load_gather(dis_l, [rv])
          dc = plsc.load_gather(dis_l, [cv])
          nb[it, pl.ds(g * L, L)] = dr * ewb[it, pl.ds(g * L, L)] * dc
      for g in range(C // L):
        nv = nb[it, pl.ds(g * L, L)]
        for j in range(L):
          ns = jnp.take_along_axis(
              nv, jnp.full((L,), j, jnp.int32), axis=0,
              mode=lax.GatherScatterMode.PROMISE_IN_BOUNDS)
          e = g * L + j
          for q in range(D // L):
            rows[b][e, pl.ds(q * L, L)] = rows[b][e, pl.ds(q * L, L)] * ns
      scatter(it, b)

    def refill(it_next, b):
      @pl.when(it_next >= NBUF)
      def _():
        pltpu.make_async_copy(rows[b], acc.at[rowb.at[it_next]],
                              ssem[b]).wait()
      gather(it_next, b)

    def loop_body(i4, carry):
      for k in range(NBUF):
        it = i4 * NBUF + k
        process(it, k)
        nxt = (k + NBUF - 1) % NBUF
        it_next = it + NBUF - 1

        @pl.when(it_next < CHUNKS)
        def _():
          refill(it_next, nxt)
      return carry
    lax.fori_loop(0, CHUNKS // NBUF, loop_body, 0)

    for k in range(NBUF):
      it = CHUNKS - NBUF + k
      b = it % NBUF
      pltpu.make_async_copy(rows[b], acc.at[rowb.at[it]], ssem[b]).wait()

    if compute_norm:
      pltpu.sync_copy(nb, norm_hbm.at[wid])

    plsc.subcore_barrier()
    pltpu.sync_copy(acc.at[pl.ds(sid * SLICE_PT, SLICE_PT)],
                    out_hbm.at[cid, pl.ds(sid * SLICE_PT, SLICE_PT)])

  return pl.kernel(body, out_type=out_types, mesh=_mesh,
                   scratch_types=scratch, compiler_params=_sc_params)


_spmm_first = _make_spmm(True)
_spmm_next = _make_spmm(False)


def _dis_body(degp_ref, dis_ref):
  deg = jnp.sum(degp_ref[...], axis=0)
  safe = jnp.maximum(deg, 1e-12)
  dis_ref[...] = jnp.where(deg > 0, lax.rsqrt(safe), 0.0)


def _dis_tc(degp):
  return pl.pallas_call(
      _dis_body,
      out_shape=jax.ShapeDtypeStruct((NP // 128, 128), jnp.float32),
  )(degp)


def _temp_body(x_ref, w1_ref, b1_ref, out_ref):
  h = jnp.dot(x_ref[...], w1_ref[...], preferred_element_type=jnp.float32)
  out_ref[...] = jnp.maximum(h + b1_ref[...], 0.0)


def _temp_tc(xp, W1, b1):
  BR = 512
  return pl.pallas_call(
      _temp_body,
      grid=(NP // BR,),
      in_specs=[
          pl.BlockSpec((BR, F_IN), lambda i: (i, 0)),
          pl.BlockSpec((F_IN, D), lambda i: (0, 0)),
          pl.BlockSpec((1, D), lambda i: (0, 0)),
      ],
      out_specs=pl.BlockSpec((BR, D), lambda i: (i, 0)),
      out_shape=jax.ShapeDtypeStruct((NP, D), jnp.float32),
  )(xp, W1, b1)


def _psum_body(p_ref, out_ref):
  out_ref[...] = p_ref[0] + p_ref[1]


def _psum_tc(partials):
  BR = 1024
  return pl.pallas_call(
      _psum_body,
      grid=(NP // BR,),
      in_specs=[pl.BlockSpec((NC, BR, D), lambda i: (0, i, 0))],
      out_specs=pl.BlockSpec((BR, D), lambda i: (i, 0)),
      out_shape=jax.ShapeDtypeStruct((NP, D), jnp.float32),
  )(partials)


def _final_body(temp_ref, s1_ref, q_ref, w2_ref, b2_ref, out_ref):
  s2 = q_ref[0] + q_ref[1]
  wa = w2_ref[0:D] + w2_ref[D:2 * D]
  wb = w2_ref[2 * D:3 * D] + w2_ref[3 * D:4 * D]
  wc = w2_ref[4 * D:5 * D]
  logits = (
      jnp.dot(temp_ref[...], wa, preferred_element_type=jnp.float32)
      + jnp.dot(s1_ref[...], wb, preferred_element_type=jnp.float32)
      + jnp.dot(s2, wc, preferred_element_type=jnp.float32)
      + b2_ref[...])
  mask = lax.broadcasted_iota(jnp.int32, logits.shape, 1) < NCLS
  logits = jnp.where(mask, logits, -jnp.inf)
  m = jnp.max(logits, axis=1, keepdims=True)
  e = jnp.where(mask, jnp.exp(logits - m), 0.0)
  lse = jnp.log(jnp.sum(e, axis=1, keepdims=True)) + m
  out_ref[...] = logits - lse


def _final_tc(temp, s1, q, W2p, b2p):
  BR = 512
  return pl.pallas_call(
      _final_body,
      grid=(NP // BR,),
      in_specs=[
          pl.BlockSpec((BR, D), lambda i: (i, 0)),
          pl.BlockSpec((BR, D), lambda i: (i, 0)),
          pl.BlockSpec((NC, BR, D), lambda i: (0, i, 0)),
          pl.BlockSpec((5 * D, NCLSP), lambda i: (0, 0)),
          pl.BlockSpec((1, NCLSP), lambda i: (0, 0)),
      ],
      out_specs=pl.BlockSpec((BR, NCLSP), lambda i: (i, 0)),
      out_shape=jax.ShapeDtypeStruct((NP, NCLSP), jnp.float32),
  )(temp, s1, q, W2p, b2p)


def kernel(x, edge_index, edge_attr, W1, b1, W2, b2):
  loop = jnp.arange(N, dtype=jnp.int32)
  row = jnp.concatenate([edge_index[0], loop])
  col = jnp.concatenate([edge_index[1], loop])
  ew = jnp.concatenate([edge_attr, jnp.ones((N,), jnp.float32)])
  pad = EP - E_REAL
  row = jnp.pad(row, (0, pad)).reshape(NW, CHUNKS, C)
  col = jnp.pad(col, (0, pad)).reshape(NW, CHUNKS, C)
  ew = jnp.pad(ew, (0, pad)).reshape(NW, CHUNKS, C)

  xp = jnp.pad(x, ((0, NP - N), (0, 0)))
  b1r = b1.reshape(1, D)
  W2p = jnp.pad(W2, ((0, 0), (0, NCLSP - NCLS)))
  b2p = jnp.pad(b2, (0, NCLSP - NCLS)).reshape(1, NCLSP)

  degp = _deg_kernel(col, ew)
  dis = _dis_tc(degp.reshape(NW, NP // 128, 128)).reshape(NP)
  temp = _temp_tc(xp, W1, b1r)

  p1, norm = _spmm_first(row, col, ew, dis, temp)
  s1 = _psum_tc(p1)
  p2 = _spmm_next(row, col, norm, s1)

  out = _final_tc(temp, s1, p2, W2p, b2p)
  return out[:N, :NCLS]


# speedup vs baseline: 1.8859x; 1.8859x over previous
import functools

import jax
import jax.numpy as jnp
from jax import lax
from jax.experimental import pallas as pl
from jax.experimental.pallas import tpu as pltpu
from jax.experimental.pallas import tpu_sc as plsc

N = 10000
NP = 10240
F_IN = 128
D = 64
NCLS = 40
NCLSP = 128

NC = 2
NS = 16
NW = NC * NS
L = 16

C = 128
NBUF = 3
E_REAL = 320000 + N
CHUNKS = NBUF * (-(-E_REAL // (NW * C * NBUF)))
EPT = CHUNKS * C
EP = EPT * NW

SLICE_PT = NP // NS

_mesh = plsc.VectorSubcoreMesh(
    core_axis_name="c", subcore_axis_name="s", num_cores=NC, num_subcores=NS)
_sc_params = pltpu.CompilerParams(
    needs_layout_passes=False, use_tc_tiling_on_sc=False)


def _worker_id():
  return lax.axis_index("s") * NC + lax.axis_index("c")


@functools.partial(
    pl.kernel,
    out_type=jax.ShapeDtypeStruct((NW, NP), jnp.float32),
    mesh=_mesh,
    scratch_types=[
        pltpu.VMEM((CHUNKS, C), jnp.int32),
        pltpu.VMEM((CHUNKS, C), jnp.float32),
        pltpu.VMEM((NP,), jnp.float32),
    ],
    compiler_params=_sc_params,
)
def _deg_kernel(col_hbm, ew_hbm, deg_hbm, colb, ewb, degl):
  wid = _worker_id()

  def zero_body(i, carry):
    degl[pl.ds(i * L, L)] = jnp.zeros((L,), jnp.float32)
    return carry
  lax.fori_loop(0, NP // L, zero_body, 0)

  pltpu.sync_copy(col_hbm.at[wid], colb)
  pltpu.sync_copy(ew_hbm.at[wid], ewb)

  def chunk_body(i, carry):
    for g in range(C // L):
      cv = colb[i, pl.ds(g * L, L)]
      ev = ewb[i, pl.ds(g * L, L)]
      plsc.addupdate_scatter(degl, [cv], ev)
    return carry
  lax.fori_loop(0, CHUNKS, chunk_body, 0)

  pltpu.sync_copy(degl, deg_hbm.at[wid])


@functools.partial(
    pl.kernel,
    out_type=jax.ShapeDtypeStruct((NC, NP, D), jnp.float32),
    mesh=_mesh,
    scratch_types=[
        pltpu.VMEM((NP,), jnp.float32),
        pltpu.VMEM((CHUNKS, C), jnp.int32),
        pltpu.VMEM((CHUNKS, C), jnp.int32),
        pltpu.VMEM((CHUNKS, C), jnp.float32),
        pltpu.VMEM((C,), jnp.float32),
        [pltpu.VMEM((C, D), jnp.bfloat16) for _ in range(NBUF)],
        [pltpu.VMEM((C, D), jnp.float32) for _ in range(NBUF)],
        pltpu.VMEM_SHARED((NP, D), jnp.float32),
        [pltpu.SemaphoreType.DMA for _ in range(NBUF)],
        [pltpu.SemaphoreType.DMA for _ in range(NBUF)],
    ],
    compiler_params=_sc_params,
)
def _spmm(row_hbm, col_hbm, ew_hbm, dis_hbm, x_hbm, out_hbm,
          dis_l, rowb, colb, ewb, nb, rows, scaled, acc, gsem, ssem):
  cid = lax.axis_index("c")
  sid = lax.axis_index("s")
  wid = sid * NC + cid

  pltpu.sync_copy(row_hbm.at[wid], rowb)
  pltpu.sync_copy(col_hbm.at[wid], colb)
  pltpu.sync_copy(ew_hbm.at[wid], ewb)
  pltpu.sync_copy(dis_hbm, dis_l)

  for r in range(C):
    for q in range(D // L):
      scaled[0][r, pl.ds(q * L, L)] = jnp.zeros((L,), jnp.float32)
  for j in range(SLICE_PT // C):
    pltpu.sync_copy(scaled[0], acc.at[pl.ds(sid * SLICE_PT + j * C, C)])
  plsc.subcore_barrier()

  def gather(it, b):
    pltpu.async_copy(x_hbm.at[colb.at[it]], rows[b], gsem[b])

  def process(it, b):
    @pl.when(it >= NBUF)
    def _():
      pltpu.make_async_copy(scaled[b], acc.at[rowb.at[it]], ssem[b]).wait()
    pltpu.make_async_copy(x_hbm.at[colb.at[it]], rows[b], gsem[b]).wait()
    for g in range(C // L):
      rv = rowb[it, pl.ds(g * L, L)]
      cv = colb[it, pl.ds(g * L, L)]
      dr = plsc.load_gather(dis_l, [rv])
      dc = plsc.load_gather(dis_l, [cv])
      nb[pl.ds(g * L, L)] = dr * ewb[it, pl.ds(g * L, L)] * dc
    for g in range(C // L):
      nv = nb[pl.ds(g * L, L)]
      for j in range(L):
        ns = jnp.take_along_axis(
            nv, jnp.full((L,), j, jnp.int32), axis=0,
            mode=lax.GatherScatterMode.PROMISE_IN_BOUNDS)
        e = g * L + j
        for h in range(D // (2 * L)):
          packed = rows[b][e, pl.ds(h * 2 * L, 2 * L)]
          lo, hi = plsc.unpack(packed, format=plsc.PackFormat.INTERLEAVED)
          scaled[b][e, pl.ds(h * 2 * L, L)] = lo * ns
          scaled[b][e, pl.ds(h * 2 * L + L, L)] = hi * ns
    pltpu.async_copy(scaled[b], acc.at[rowb.at[it]], ssem[b], add=True)

  for b in range(NBUF - 1):
    gather(b, b)

  def loop_body(i4, carry):
    for k in range(NBUF):
      it = i4 * NBUF + k
      process(it, k)
      nxt = (k + NBUF - 1) % NBUF
      it_next = it + NBUF - 1

      @pl.when(it_next < CHUNKS)
      def _():
        gather(it_next, nxt)
    return carry
  lax.fori_loop(0, CHUNKS // NBUF, loop_body, 0)

  for k in range(NBUF):
    it = CHUNKS - NBUF + k
    b = it % NBUF
    pltpu.make_async_copy(scaled[b], acc.at[rowb.at[it]], ssem[b]).wait()

  plsc.subcore_barrier()
  for j in range(SLICE_PT // C):
    base = sid * SLICE_PT + j * C
    pltpu.sync_copy(acc.at[pl.ds(base, C)], scaled[0])
    pltpu.sync_copy(scaled[0], out_hbm.at[cid, pl.ds(base, C)])


def _interleave_bf16(t):
  r = t.shape[0]
  ti = jnp.swapaxes(t.reshape(r, D // 32, 2, L), 2, 3).reshape(r, D)
  return ti.astype(jnp.bfloat16)


def _dis_body(degp_ref, dis_ref):
  deg = jnp.sum(degp_ref[...], axis=0)
  safe = jnp.maximum(deg, 1e-12)
  dis_ref[...] = jnp.where(deg > 0, lax.rsqrt(safe), 0.0)


def _dis_tc(degp):
  return pl.pallas_call(
      _dis_body,
      out_shape=jax.ShapeDtypeStruct((NP // 128, 128), jnp.float32),
  )(degp)


def _temp_body(x_ref, w1_ref, b1_ref, out_ref, outb_ref):
  h = jnp.dot(x_ref[...], w1_ref[...], preferred_element_type=jnp.float32)
  t = jnp.maximum(h + b1_ref[...], 0.0)
  out_ref[...] = t
  outb_ref[...] = _interleave_bf16(t)


def _temp_tc(xp, W1, b1):
  BR = 512
  return pl.pallas_call(
      _temp_body,
      grid=(NP // BR,),
      in_specs=[
          pl.BlockSpec((BR, F_IN), lambda i: (i, 0)),
          pl.BlockSpec((F_IN, D), lambda i: (0, 0)),
          pl.BlockSpec((1, D), lambda i: (0, 0)),
      ],
      out_specs=[pl.BlockSpec((BR, D), lambda i: (i, 0)),
                 pl.BlockSpec((BR, D), lambda i: (i, 0))],
      out_shape=[jax.ShapeDtypeStruct((NP, D), jnp.float32),
                 jax.ShapeDtypeStruct((NP, D), jnp.bfloat16)],
  )(xp, W1, b1)


def _psum_body(p_ref, out_ref, outb_ref):
  t = p_ref[0] + p_ref[1]
  out_ref[...] = t
  outb_ref[...] = _interleave_bf16(t)


def _psum_tc(partials):
  BR = 1024
  return pl.pallas_call(
      _psum_body,
      grid=(NP // BR,),
      in_specs=[pl.BlockSpec((NC, BR, D), lambda i: (0, i, 0))],
      out_specs=[pl.BlockSpec((BR, D), lambda i: (i, 0)),
                 pl.BlockSpec((BR, D), lambda i: (i, 0))],
      out_shape=[jax.ShapeDtypeStruct((NP, D), jnp.float32),
                 jax.ShapeDtypeStruct((NP, D), jnp.bfloat16)],
  )(partials)


def _final_body(temp_ref, s1_ref, q_ref, w2_ref, b2_ref, out_ref):
  s2 = q_ref[0] + q_ref[1]
  wa = w2_ref[0:D] + w2_ref[D:2 * D]
  wb = w2_ref[2 * D:3 * D] + w2_ref[3 * D:4 * D]
  wc = w2_ref[4 * D:5 * D]
  logits = (
      jnp.dot(temp_ref[...], wa, preferred_element_type=jnp.float32)
      + jnp.dot(s1_ref[...], wb, preferred_element_type=jnp.float32)
      + jnp.dot(s2, wc, preferred_element_type=jnp.float32)
      + b2_ref[...])
  mask = lax.broadcasted_iota(jnp.int32, logits.shape, 1) < NCLS
  logits = jnp.where(mask, logits, -jnp.inf)
  m = jnp.max(logits, axis=1, keepdims=True)
  e = jnp.where(mask, jnp.exp(logits - m), 0.0)
  lse = jnp.log(jnp.sum(e, axis=1, keepdims=True)) + m
  out_ref[...] = logits - lse


def _final_tc(temp, s1, q, W2p, b2p):
  BR = 512
  return pl.pallas_call(
      _final_body,
      grid=(NP // BR,),
      in_specs=[
          pl.BlockSpec((BR, D), lambda i: (i, 0)),
          pl.BlockSpec((BR, D), lambda i: (i, 0)),
          pl.BlockSpec((NC, BR, D), lambda i: (0, i, 0)),
          pl.BlockSpec((5 * D, NCLSP), lambda i: (0, 0)),
          pl.BlockSpec((1, NCLSP), lambda i: (0, 0)),
      ],
      out_specs=pl.BlockSpec((BR, NCLSP), lambda i: (i, 0)),
      out_shape=jax.ShapeDtypeStruct((NP, NCLSP), jnp.float32),
  )(temp, s1, q, W2p, b2p)


def kernel(x, edge_index, edge_attr, W1, b1, W2, b2):
  loop = jnp.arange(N, dtype=jnp.int32)
  row = jnp.concatenate([edge_index[0], loop])
  col = jnp.concatenate([edge_index[1], loop])
  ew = jnp.concatenate([edge_attr, jnp.ones((N,), jnp.float32)])
  pad = EP - E_REAL
  row = jnp.pad(row, (0, pad)).reshape(NW, CHUNKS, C)
  col = jnp.pad(col, (0, pad)).reshape(NW, CHUNKS, C)
  ew = jnp.pad(ew, (0, pad)).reshape(NW, CHUNKS, C)

  xp = jnp.pad(x, ((0, NP - N), (0, 0)))
  b1r = b1.reshape(1, D)
  W2p = jnp.pad(W2, ((0, 0), (0, NCLSP - NCLS)))
  b2p = jnp.pad(b2, (0, NCLSP - NCLS)).reshape(1, NCLSP)

  degp = _deg_kernel(col, ew)
  dis = _dis_tc(degp.reshape(NW, NP // 128, 128)).reshape(NP)
  temp, temp_b = _temp_tc(xp, W1, b1r)

  p1 = _spmm(row, col, ew, dis, temp_b)
  s1, s1_b = _psum_tc(p1)
  p2 = _spmm(row, col, ew, dis, s1_b)

  out = _final_tc(temp, s1, p2, W2p, b2p)
  return out[:N, :NCLS]
